# baseline (speedup 1.0000x reference)
import math

import jax
import jax.numpy as jnp
from jax.experimental import pallas as pl
from jax.experimental.pallas import tpu as pltpu

_HD = 128
_T = 4096
_B = 16
_PL0 = 256
_PN0 = 16
_PN1 = 4
_QL = 1024
_NQ = _T // _QL
_PPQ = _QL // _PL0
_IR3 = 1.0 / math.sqrt(3.0)

_D1 = (_IR3, 0.5, 0.5, _IR3)
_U1 = tuple(
    _D1[j] * ((_D1[j - 1] if j > 0 else 0.0) + 2.0 * _D1[j] + (_D1[j + 1] if j < 3 else 0.0)) / 4.0
    for j in range(4)
)


def _body(len_ref, x_ref, w0_ref, b0_ref, w1_ref, b1_ref, w2_ref, b2_ref,
          o_ref, s_ref):
    b = pl.program_id(0)
    q = pl.program_id(1)
    ln = len_ref[b]

    @pl.when(ln > q * _QL)
    def _compute_quarter():
        xb = x_ref[0]
        j = jax.lax.broadcasted_iota(jnp.int32, (1, _QL), 1) + q * _QL
        p = j // _PL0
        jj = j - p * _PL0
        m = jnp.clip(ln - p * _PL0, 0, _PL0)
        half = jnp.float32(0.5)
        ir3 = jnp.float32(_IR3)
        dd = jnp.where((jj == 0) | (jj == _PL0 - 1), ir3, half)
        dm1 = jnp.where(jj == 1, ir3, half)
        dp1 = jnp.where(jj == _PL0 - 2, ir3, half)
        gp = ((jj >= 1) & (jj <= m)).astype(jnp.float32)
        gs = (jj < m).astype(jnp.float32)
        gn = ((jj <= _PL0 - 2) & (jj + 1 < m)).astype(jnp.float32)
        w = dd * (dm1 * gp + 2.0 * dd * gs + dp1 * gn)
        w = w / jnp.maximum(m.astype(jnp.float32), 1.0)
        xw = xb * w
        cols = [
            jnp.sum(xw[:, k * _PL0:(k + 1) * _PL0], axis=1, keepdims=True)
            for k in range(_PPQ)
        ]
        s_ref[q] = jnp.concatenate(cols, axis=1)

    @pl.when(ln <= q * _QL)
    def _zero_quarter():
        s_ref[q] = jnp.zeros((_HD, _PPQ), jnp.float32)

    @pl.when(q == _NQ - 1)
    def _finish():
        s0 = jnp.concatenate([s_ref[i] for i in range(_NQ)], axis=1)
        h0 = jax.lax.dot_general(
            w0_ref[...], s0, (((0,), (0,)), ((), ())),
            preferred_element_type=jnp.float32)
        pidx = jax.lax.broadcasted_iota(jnp.int32, (1, _PN0), 1)
        gate = (ln > pidx * _PL0).astype(jnp.float32)
        h0 = h0 + b0_ref[...] * gate

        cols1 = [
            _U1[0] * h0[:, 4 * t:4 * t + 1]
            + _U1[1] * h0[:, 4 * t + 1:4 * t + 2]
            + _U1[2] * h0[:, 4 * t + 2:4 * t + 3]
            + _U1[3] * h0[:, 4 * t + 3:4 * t + 4]
            for t in range(_PN1)
        ]
        s1 = jnp.concatenate(cols1, axis=1)

        h1 = jax.lax.dot_general(
            w1_ref[...], s1, (((0,), (0,)), ((), ())),
            preferred_element_type=jnp.float32) + b1_ref[...]
        out = jax.lax.dot_general(
            w2_ref[...], h1, (((0,), (0,)), ((), ())),
            preferred_element_type=jnp.float32) + b2_ref[...]
        o_ref[0] = out


def _x_index(b, q, L):
    nq_last = jnp.maximum((L[b] + _QL - 1) // _QL - 1, 0)
    return (b, 0, jnp.minimum(q, nq_last))


def kernel(x, lengths, W0, b0, W1, b1, W2, b2):
    b0c = b0.reshape(_HD, 1)
    b1c = b1.reshape(_HD, 1)
    b2c = b2.reshape(_HD, 1)
    wspec = pl.BlockSpec((_HD, _HD), lambda b, q, L: (0, 0))
    bspec = pl.BlockSpec((_HD, 1), lambda b, q, L: (0, 0))
    return pl.pallas_call(
        _body,
        grid_spec=pltpu.PrefetchScalarGridSpec(
            num_scalar_prefetch=1,
            grid=(_B, _NQ),
            in_specs=[
                pl.BlockSpec((1, _HD, _QL), _x_index),
                wspec, bspec, wspec, bspec, wspec, bspec,
            ],
            out_specs=pl.BlockSpec((1, _HD, _PN1), lambda b, q, L: (b, 0, 0)),
            scratch_shapes=[pltpu.VMEM((_NQ, _HD, _PPQ), jnp.float32)],
        ),
        out_shape=jax.ShapeDtypeStruct((_B, _HD, _PN1), jnp.float32),
        compiler_params=pltpu.CompilerParams(
            dimension_semantics=("arbitrary", "arbitrary")),
    )(lengths, x, W0, b0c, W1, b1c, W2, b2c)


# speedup vs baseline: 1.0463x; 1.0463x over previous
import math

import jax
import jax.numpy as jnp
from jax.experimental import pallas as pl
from jax.experimental.pallas import tpu as pltpu

_HD = 128
_T = 4096
_B = 16
_PL0 = 256
_PN0 = 16
_PN1 = 4
_QL = 1024
_NQ = _T // _QL
_PPQ = _QL // _PL0
_NSTEP = _B * _NQ
_IR3 = 1.0 / math.sqrt(3.0)

_D1 = (_IR3, 0.5, 0.5, _IR3)
_U1 = tuple(
    _D1[j] * ((_D1[j - 1] if j > 0 else 0.0) + 2.0 * _D1[j] + (_D1[j + 1] if j < 3 else 0.0)) / 4.0
    for j in range(4)
)


def _body(len_ref, x_ref, w0_ref, b0_ref, w1_ref, b1_ref, w2_ref, b2_ref,
          o_ref, buf_ref, s_ref, sem_ref):

    def _issue(t):
        b = t // _NQ
        q = t % _NQ
        slot = t % 2
        pltpu.make_async_copy(
            x_ref.at[b, :, pl.ds(q * _QL, _QL)],
            buf_ref.at[slot],
            sem_ref.at[slot],
        ).start()

    @pl.when(len_ref[0] > 0)
    def _prologue():
        _issue(0)

    def _step(t, carry):
        b = t // _NQ
        q = t % _NQ
        ln = len_ref[b]
        valid = ln > q * _QL

        nt = t + 1

        @pl.when(nt < _NSTEP)
        def _issue_next():
            @pl.when(len_ref[nt // _NQ] > (nt % _NQ) * _QL)
            def _():
                _issue(nt)

        @pl.when(valid)
        def _compute_quarter():
            slot = t % 2
            pltpu.make_async_copy(
                x_ref.at[b, :, pl.ds(q * _QL, _QL)],
                buf_ref.at[slot],
                sem_ref.at[slot],
            ).wait()
            xb = buf_ref[slot]
            j = jax.lax.broadcasted_iota(jnp.int32, (1, _QL), 1) + q * _QL
            p = j // _PL0
            jj = j - p * _PL0
            m = jnp.clip(ln - p * _PL0, 0, _PL0)
            half = jnp.float32(0.5)
            ir3 = jnp.float32(_IR3)
            dd = jnp.where((jj == 0) | (jj == _PL0 - 1), ir3, half)
            dm1 = jnp.where(jj == 1, ir3, half)
            dp1 = jnp.where(jj == _PL0 - 2, ir3, half)
            gp = ((jj >= 1) & (jj <= m)).astype(jnp.float32)
            gs = (jj < m).astype(jnp.float32)
            gn = ((jj <= _PL0 - 2) & (jj + 1 < m)).astype(jnp.float32)
            w = dd * (dm1 * gp + 2.0 * dd * gs + dp1 * gn)
            w = w / jnp.maximum(m.astype(jnp.float32), 1.0)
            xw = xb * w
            cols = [
                jnp.sum(xw[:, k * _PL0:(k + 1) * _PL0], axis=1, keepdims=True)
                for k in range(_PPQ)
            ]
            s_ref[q] = jnp.concatenate(cols, axis=1)

        @pl.when(jnp.logical_not(valid))
        def _zero_quarter():
            s_ref[q] = jnp.zeros((_HD, _PPQ), jnp.float32)

        @pl.when(q == _NQ - 1)
        def _finish_batch():
            s0 = jnp.concatenate([s_ref[i] for i in range(_NQ)], axis=1)
            h0 = jax.lax.dot_general(
                w0_ref[...], s0, (((0,), (0,)), ((), ())),
                preferred_element_type=jnp.float32)
            pidx = jax.lax.broadcasted_iota(jnp.int32, (1, _PN0), 1)
            gate = (ln > pidx * _PL0).astype(jnp.float32)
            h0 = h0 + b0_ref[...] * gate

            cols1 = [
                _U1[0] * h0[:, 4 * u:4 * u + 1]
                + _U1[1] * h0[:, 4 * u + 1:4 * u + 2]
                + _U1[2] * h0[:, 4 * u + 2:4 * u + 3]
                + _U1[3] * h0[:, 4 * u + 3:4 * u + 4]
                for u in range(_PN1)
            ]
            s1 = jnp.concatenate(cols1, axis=1)

            h1 = jax.lax.dot_general(
                w1_ref[...], s1, (((0,), (0,)), ((), ())),
                preferred_element_type=jnp.float32) + b1_ref[...]
            out = jax.lax.dot_general(
                w2_ref[...], h1, (((0,), (0,)), ((), ())),
                preferred_element_type=jnp.float32) + b2_ref[...]
            o_ref[b] = out

        return carry

    jax.lax.fori_loop(0, _NSTEP, _step, 0)


def kernel(x, lengths, W0, b0, W1, b1, W2, b2):
    b0c = b0.reshape(_HD, 1)
    b1c = b1.reshape(_HD, 1)
    b2c = b2.reshape(_HD, 1)
    wspec = pl.BlockSpec((_HD, _HD), lambda i, L: (0, 0))
    bspec = pl.BlockSpec((_HD, 1), lambda i, L: (0, 0))
    return pl.pallas_call(
        _body,
        grid_spec=pltpu.PrefetchScalarGridSpec(
            num_scalar_prefetch=1,
            grid=(1,),
            in_specs=[
                pl.BlockSpec(memory_space=pltpu.MemorySpace.HBM),
                wspec, bspec, wspec, bspec, wspec, bspec,
            ],
            out_specs=pl.BlockSpec(
                (_B, _HD, _PN1), lambda i, L: (0, 0, 0)),
            scratch_shapes=[
                pltpu.VMEM((2, _HD, _QL), jnp.float32),
                pltpu.VMEM((_NQ, _HD, _PPQ), jnp.float32),
                pltpu.SemaphoreType.DMA((2,)),
            ],
        ),
        out_shape=jax.ShapeDtypeStruct((_B, _HD, _PN1), jnp.float32),
    )(lengths, x, W0, b0c, W1, b1c, W2, b2c)


# speedup vs baseline: 1.4244x; 1.3614x over previous
import math

import jax
import jax.numpy as jnp
from jax.experimental import pallas as pl
from jax.experimental.pallas import tpu as pltpu

_HD = 128
_T = 4096
_B = 16
_PL0 = 256
_PN0 = 16
_PN1 = 4
_QL = 1024
_NQ = _T // _QL
_PPQ = _QL // _PL0
_IR3 = 1.0 / math.sqrt(3.0)

_D1 = (_IR3, 0.5, 0.5, _IR3)
_U1 = tuple(
    _D1[j] * ((_D1[j - 1] if j > 0 else 0.0) + 2.0 * _D1[j] + (_D1[j + 1] if j < 3 else 0.0)) / 4.0
    for j in range(4)
)


def _body(len_ref, x_ref, w0_ref, b0_ref, w1_ref, b1_ref, w2_ref, b2_ref,
          o_ref, buf_ref, s_ref, sem_ref):

    def _nq_of(ln):
        return (ln + _QL - 1) // _QL

    def _copy(b, slot, k):
        return pltpu.make_async_copy(
            x_ref.at[b, :, pl.ds(0, k * _QL)],
            buf_ref.at[slot, :, pl.ds(0, k * _QL)],
            sem_ref.at[slot],
        )

    def _issue(b, slot):
        k = _nq_of(len_ref[b])
        for kk in range(1, _NQ + 1):
            @pl.when(k == kk)
            def _():
                _copy(b, slot, kk).start()

    def _wait(b, slot):
        k = _nq_of(len_ref[b])
        for kk in range(1, _NQ + 1):
            @pl.when(k == kk)
            def _():
                _copy(b, slot, kk).wait()

    @pl.when(len_ref[0] > 0)
    def _prologue():
        _issue(0, 0)

    def _step(b, carry):
        ln = len_ref[b]
        slot = b % 2

        @pl.when(b + 1 < _B)
        def _issue_next():
            @pl.when(len_ref[b + 1] > 0)
            def _():
                _issue(b + 1, (b + 1) % 2)

        _wait(b, slot)

        for q in range(_NQ):
            @pl.when(ln > q * _QL)
            def _compute_quarter(q=q):
                xq = buf_ref[slot][:, q * _QL:(q + 1) * _QL]
                j = jax.lax.broadcasted_iota(jnp.int32, (1, _QL), 1) + q * _QL
                p = j // _PL0
                jj = j - p * _PL0
                m = jnp.clip(ln - p * _PL0, 0, _PL0)
                half = jnp.float32(0.5)
                ir3 = jnp.float32(_IR3)
                dd = jnp.where((jj == 0) | (jj == _PL0 - 1), ir3, half)
                dm1 = jnp.where(jj == 1, ir3, half)
                dp1 = jnp.where(jj == _PL0 - 2, ir3, half)
                gp = ((jj >= 1) & (jj <= m)).astype(jnp.float32)
                gs = (jj < m).astype(jnp.float32)
                gn = ((jj <= _PL0 - 2) & (jj + 1 < m)).astype(jnp.float32)
                w = dd * (dm1 * gp + 2.0 * dd * gs + dp1 * gn)
                w = w / jnp.maximum(m.astype(jnp.float32), 1.0)
                xw = xq * w
                cols = [
                    jnp.sum(xw[:, kk * _PL0:(kk + 1) * _PL0], axis=1, keepdims=True)
                    for kk in range(_PPQ)
                ]
                s_ref[q] = jnp.concatenate(cols, axis=1)

            @pl.when(ln <= q * _QL)
            def _zero_quarter(q=q):
                s_ref[q] = jnp.zeros((_HD, _PPQ), jnp.float32)

        s0 = jnp.concatenate([s_ref[i] for i in range(_NQ)], axis=1)

        h0 = jax.lax.dot_general(
            w0_ref[...], s0, (((0,), (0,)), ((), ())),
            preferred_element_type=jnp.float32)
        pidx = jax.lax.broadcasted_iota(jnp.int32, (1, _PN0), 1)
        gate = (ln > pidx * _PL0).astype(jnp.float32)
        h0 = h0 + b0_ref[...] * gate

        cols1 = [
            _U1[0] * h0[:, 4 * u:4 * u + 1]
            + _U1[1] * h0[:, 4 * u + 1:4 * u + 2]
            + _U1[2] * h0[:, 4 * u + 2:4 * u + 3]
            + _U1[3] * h0[:, 4 * u + 3:4 * u + 4]
            for u in range(_PN1)
        ]
        s1 = jnp.concatenate(cols1, axis=1)

        h1 = jax.lax.dot_general(
            w1_ref[...], s1, (((0,), (0,)), ((), ())),
            preferred_element_type=jnp.float32) + b1_ref[...]
        out = jax.lax.dot_general(
            w2_ref[...], h1, (((0,), (0,)), ((), ())),
            preferred_element_type=jnp.float32) + b2_ref[...]
        o_ref[b] = out
        return carry

    jax.lax.fori_loop(0, _B, _step, 0)


def kernel(x, lengths, W0, b0, W1, b1, W2, b2):
    b0c = b0.reshape(_HD, 1)
    b1c = b1.reshape(_HD, 1)
    b2c = b2.reshape(_HD, 1)
    wspec = pl.BlockSpec((_HD, _HD), lambda i, L: (0, 0))
    bspec = pl.BlockSpec((_HD, 1), lambda i, L: (0, 0))
    return pl.pallas_call(
        _body,
        grid_spec=pltpu.PrefetchScalarGridSpec(
            num_scalar_prefetch=1,
            grid=(1,),
            in_specs=[
                pl.BlockSpec(memory_space=pltpu.MemorySpace.HBM),
                wspec, bspec, wspec, bspec, wspec, bspec,
            ],
            out_specs=pl.BlockSpec(
                (_B, _HD, _PN1), lambda i, L: (0, 0, 0)),
            scratch_shapes=[
                pltpu.VMEM((2, _HD, _T), jnp.float32),
                pltpu.VMEM((_NQ, _HD, _PPQ), jnp.float32),
                pltpu.SemaphoreType.DMA((2,)),
            ],
        ),
        out_shape=jax.ShapeDtypeStruct((_B, _HD, _PN1), jnp.float32),
    )(lengths, x, W0, b0c, W1, b1c, W2, b2c)


# speedup vs baseline: 1.5365x; 1.0787x over previous
import math

import jax
import jax.numpy as jnp
from jax.experimental import pallas as pl
from jax.experimental.pallas import tpu as pltpu

_HD = 128
_T = 4096
_B = 16
_PL0 = 256
_PN0 = 16
_PN1 = 4
_QL = 1024
_NQ = _T // _QL
_PPQ = _QL // _PL0
_NBUF = 4
_IR3 = 1.0 / math.sqrt(3.0)

_D1 = (_IR3, 0.5, 0.5, _IR3)
_U1 = tuple(
    _D1[j] * ((_D1[j - 1] if j > 0 else 0.0) + 2.0 * _D1[j] + (_D1[j + 1] if j < 3 else 0.0)) / 4.0
    for j in range(4)
)


def _body(len_ref, x_ref, w0_ref, b0_ref, w1_ref, b1_ref, w2_ref, b2_ref,
          o_ref, buf_ref, s_ref, sem_ref):

    def _nq_of(ln):
        return (ln + _QL - 1) // _QL

    def _copy(b, slot, k):
        return pltpu.make_async_copy(
            x_ref.at[b, :, pl.ds(0, k * _QL)],
            buf_ref.at[slot, :, pl.ds(0, k * _QL)],
            sem_ref.at[slot],
        )

    def _issue(b, slot):
        k = _nq_of(len_ref[b])
        for kk in range(1, _NQ + 1):
            @pl.when(k == kk)
            def _():
                _copy(b, slot, kk).start()

    def _wait(b, slot):
        k = _nq_of(len_ref[b])
        for kk in range(1, _NQ + 1):
            @pl.when(k == kk)
            def _():
                _copy(b, slot, kk).wait()

    for i in range(_NBUF - 1):
        @pl.when(len_ref[i] > 0)
        def _prologue(i=i):
            _issue(i, i)

    def _step(b, carry):
        ln = len_ref[b]
        slot = b % _NBUF

        @pl.when(b + _NBUF - 1 < _B)
        def _issue_next():
            @pl.when(len_ref[b + _NBUF - 1] > 0)
            def _():
                _issue(b + _NBUF - 1, (b + _NBUF - 1) % _NBUF)

        _wait(b, slot)

        for q in range(_NQ):
            @pl.when(ln > q * _QL)
            def _compute_quarter(q=q):
                xq = buf_ref[slot][:, q * _QL:(q + 1) * _QL]
                j = jax.lax.broadcasted_iota(jnp.int32, (1, _QL), 1) + q * _QL
                p = j // _PL0
                jj = j - p * _PL0
                m = jnp.clip(ln - p * _PL0, 0, _PL0)
                half = jnp.float32(0.5)
                ir3 = jnp.float32(_IR3)
                dd = jnp.where((jj == 0) | (jj == _PL0 - 1), ir3, half)
                dm1 = jnp.where(jj == 1, ir3, half)
                dp1 = jnp.where(jj == _PL0 - 2, ir3, half)
                gp = ((jj >= 1) & (jj <= m)).astype(jnp.float32)
                gs = (jj < m).astype(jnp.float32)
                gn = ((jj <= _PL0 - 2) & (jj + 1 < m)).astype(jnp.float32)
                w = dd * (dm1 * gp + 2.0 * dd * gs + dp1 * gn)
                w = w / jnp.maximum(m.astype(jnp.float32), 1.0)
                xw = xq * w
                cols = [
                    jnp.sum(xw[:, kk * _PL0:(kk + 1) * _PL0], axis=1, keepdims=True)
                    for kk in range(_PPQ)
                ]
                s_ref[q] = jnp.concatenate(cols, axis=1)

            @pl.when(ln <= q * _QL)
            def _zero_quarter(q=q):
                s_ref[q] = jnp.zeros((_HD, _PPQ), jnp.float32)

        s0 = jnp.concatenate([s_ref[i] for i in range(_NQ)], axis=1)

        h0 = jax.lax.dot_general(
            w0_ref[...], s0, (((0,), (0,)), ((), ())),
            preferred_element_type=jnp.float32)
        pidx = jax.lax.broadcasted_iota(jnp.int32, (1, _PN0), 1)
        gate = (ln > pidx * _PL0).astype(jnp.float32)
        h0 = h0 + b0_ref[...] * gate

        cols1 = [
            _U1[0] * h0[:, 4 * u:4 * u + 1]
            + _U1[1] * h0[:, 4 * u + 1:4 * u + 2]
            + _U1[2] * h0[:, 4 * u + 2:4 * u + 3]
            + _U1[3] * h0[:, 4 * u + 3:4 * u + 4]
            for u in range(_PN1)
        ]
        s1 = jnp.concatenate(cols1, axis=1)

        h1 = jax.lax.dot_general(
            w1_ref[...], s1, (((0,), (0,)), ((), ())),
            preferred_element_type=jnp.float32) + b1_ref[...]
        out = jax.lax.dot_general(
            w2_ref[...], h1, (((0,), (0,)), ((), ())),
            preferred_element_type=jnp.float32) + b2_ref[...]
        o_ref[b] = out
        return carry

    jax.lax.fori_loop(0, _B, _step, 0)


def kernel(x, lengths, W0, b0, W1, b1, W2, b2):
    b0c = b0.reshape(_HD, 1)
    b1c = b1.reshape(_HD, 1)
    b2c = b2.reshape(_HD, 1)
    wspec = pl.BlockSpec((_HD, _HD), lambda i, L: (0, 0))
    bspec = pl.BlockSpec((_HD, 1), lambda i, L: (0, 0))
    return pl.pallas_call(
        _body,
        grid_spec=pltpu.PrefetchScalarGridSpec(
            num_scalar_prefetch=1,
            grid=(1,),
            in_specs=[
                pl.BlockSpec(memory_space=pltpu.MemorySpace.HBM),
                wspec, bspec, wspec, bspec, wspec, bspec,
            ],
            out_specs=pl.BlockSpec(
                (_B, _HD, _PN1), lambda i, L: (0, 0, 0)),
            scratch_shapes=[
                pltpu.VMEM((_NBUF, _HD, _T), jnp.float32),
                pltpu.VMEM((_NQ, _HD, _PPQ), jnp.float32),
                pltpu.SemaphoreType.DMA((_NBUF,)),
            ],
        ),
        out_shape=jax.ShapeDtypeStruct((_B, _HD, _PN1), jnp.float32),
    )(lengths, x, W0, b0c, W1, b1c, W2, b2c)


# speedup vs baseline: 1.5385x; 1.0014x over previous
import math

import jax
import jax.numpy as jnp
from jax.experimental import pallas as pl
from jax.experimental.pallas import tpu as pltpu

_HD = 128
_T = 4096
_B = 16
_PL0 = 256
_PN0 = 16
_PN1 = 4
_QL = 1024
_NQ = _T // _QL
_PPQ = _QL // _PL0
_NBUF = 8
_IR3 = 1.0 / math.sqrt(3.0)

_D1 = (_IR3, 0.5, 0.5, _IR3)
_U1 = tuple(
    _D1[j] * ((_D1[j - 1] if j > 0 else 0.0) + 2.0 * _D1[j] + (_D1[j + 1] if j < 3 else 0.0)) / 4.0
    for j in range(4)
)


def _body(len_ref, x_ref, w0_ref, b0_ref, w1_ref, b1_ref, w2_ref, b2_ref,
          o_ref, buf_ref, s_ref, sem_ref):

    def _nq_of(ln):
        return (ln + _QL - 1) // _QL

    def _copy(b, slot, k):
        return pltpu.make_async_copy(
            x_ref.at[b, :, pl.ds(0, k * _QL)],
            buf_ref.at[slot, :, pl.ds(0, k * _QL)],
            sem_ref.at[slot],
        )

    def _issue(b, slot):
        k = _nq_of(len_ref[b])
        for kk in range(1, _NQ + 1):
            @pl.when(k == kk)
            def _():
                _copy(b, slot, kk).start()

    def _wait(b, slot):
        k = _nq_of(len_ref[b])
        for kk in range(1, _NQ + 1):
            @pl.when(k == kk)
            def _():
                _copy(b, slot, kk).wait()

    for i in range(_NBUF - 1):
        @pl.when(len_ref[i] > 0)
        def _prologue(i=i):
            _issue(i, i)

    def _step(b, carry):
        ln = len_ref[b]
        slot = b % _NBUF

        @pl.when(b + _NBUF - 1 < _B)
        def _issue_next():
            @pl.when(len_ref[b + _NBUF - 1] > 0)
            def _():
                _issue(b + _NBUF - 1, (b + _NBUF - 1) % _NBUF)

        _wait(b, slot)

        for q in range(_NQ):
            @pl.when(ln > q * _QL)
            def _compute_quarter(q=q):
                xq = buf_ref[slot][:, q * _QL:(q + 1) * _QL]
                j = jax.lax.broadcasted_iota(jnp.int32, (1, _QL), 1) + q * _QL
                p = j // _PL0
                jj = j - p * _PL0
                m = jnp.clip(ln - p * _PL0, 0, _PL0)
                half = jnp.float32(0.5)
                ir3 = jnp.float32(_IR3)
                dd = jnp.where((jj == 0) | (jj == _PL0 - 1), ir3, half)
                dm1 = jnp.where(jj == 1, ir3, half)
                dp1 = jnp.where(jj == _PL0 - 2, ir3, half)
                gp = ((jj >= 1) & (jj <= m)).astype(jnp.float32)
                gs = (jj < m).astype(jnp.float32)
                gn = ((jj <= _PL0 - 2) & (jj + 1 < m)).astype(jnp.float32)
                w = dd * (dm1 * gp + 2.0 * dd * gs + dp1 * gn)
                w = w / jnp.maximum(m.astype(jnp.float32), 1.0)
                xw = xq * w
                cols = [
                    jnp.sum(xw[:, kk * _PL0:(kk + 1) * _PL0], axis=1, keepdims=True)
                    for kk in range(_PPQ)
                ]
                s_ref[q] = jnp.concatenate(cols, axis=1)

            @pl.when(ln <= q * _QL)
            def _zero_quarter(q=q):
                s_ref[q] = jnp.zeros((_HD, _PPQ), jnp.float32)

        s0 = jnp.concatenate([s_ref[i] for i in range(_NQ)], axis=1)

        h0 = jax.lax.dot_general(
            w0_ref[...], s0, (((0,), (0,)), ((), ())),
            preferred_element_type=jnp.float32)
        pidx = jax.lax.broadcasted_iota(jnp.int32, (1, _PN0), 1)
        gate = (ln > pidx * _PL0).astype(jnp.float32)
        h0 = h0 + b0_ref[...] * gate

        cols1 = [
            _U1[0] * h0[:, 4 * u:4 * u + 1]
            + _U1[1] * h0[:, 4 * u + 1:4 * u + 2]
            + _U1[2] * h0[:, 4 * u + 2:4 * u + 3]
            + _U1[3] * h0[:, 4 * u + 3:4 * u + 4]
            for u in range(_PN1)
        ]
        s1 = jnp.concatenate(cols1, axis=1)

        h1 = jax.lax.dot_general(
            w1_ref[...], s1, (((0,), (0,)), ((), ())),
            preferred_element_type=jnp.float32) + b1_ref[...]
        out = jax.lax.dot_general(
            w2_ref[...], h1, (((0,), (0,)), ((), ())),
            preferred_element_type=jnp.float32) + b2_ref[...]
        o_ref[b] = out
        return carry

    jax.lax.fori_loop(0, _B, _step, 0)


def kernel(x, lengths, W0, b0, W1, b1, W2, b2):
    b0c = b0.reshape(_HD, 1)
    b1c = b1.reshape(_HD, 1)
    b2c = b2.reshape(_HD, 1)
    wspec = pl.BlockSpec((_HD, _HD), lambda i, L: (0, 0))
    bspec = pl.BlockSpec((_HD, 1), lambda i, L: (0, 0))
    return pl.pallas_call(
        _body,
        grid_spec=pltpu.PrefetchScalarGridSpec(
            num_scalar_prefetch=1,
            grid=(1,),
            in_specs=[
                pl.BlockSpec(memory_space=pltpu.MemorySpace.HBM),
                wspec, bspec, wspec, bspec, wspec, bspec,
            ],
            out_specs=pl.BlockSpec(
                (_B, _HD, _PN1), lambda i, L: (0, 0, 0)),
            scratch_shapes=[
                pltpu.VMEM((_NBUF, _HD, _T), jnp.float32),
                pltpu.VMEM((_NQ, _HD, _PPQ), jnp.float32),
                pltpu.SemaphoreType.DMA((_NBUF,)),
            ],
        ),
        out_shape=jax.ShapeDtypeStruct((_B, _HD, _PN1), jnp.float32),
    )(lengths, x, W0, b0c, W1, b1c, W2, b2c)
